# baseline (speedup 1.0000x reference)
import functools

import jax
import jax.numpy as jnp
import numpy as np
from jax import lax
from jax.experimental import pallas as pl
from jax.experimental.pallas import tpu as pltpu
from jax.experimental.pallas import tpu_sc as plsc

N = 10000
E = 160000
M = 1000
D = 256
EMBED_DIM = 256
NC, NS, L = 2, 16, 16
NW = NC * NS

DV = 144

CHUNK = 128
NCHUNK = E // CHUNK
K2_ITERS = (NCHUNK + NW - 1) // NW
K3_ITERS = (NCHUNK + NS - 1) // NS

_mesh = plsc.VectorSubcoreMesh(core_axis_name="c", subcore_axis_name="s")
_sc_params = pltpu.CompilerParams(use_tc_tiling_on_sc=False)


def _matmul_bias(x, w_t, b, block_n):
    n, k = x.shape
    o = w_t.shape[1]

    def body(xr, wr, br, yr):
        yr[...] = (
            jnp.dot(xr[...], wr[...], preferred_element_type=jnp.float32)
            + br[...]
        )

    return pl.pallas_call(
        body,
        grid=(n // block_n,),
        in_specs=[
            pl.BlockSpec((block_n, k), lambda i: (i, 0)),
            pl.BlockSpec((k, o), lambda i: (0, 0)),
            pl.BlockSpec((1, o), lambda i: (0, 0)),
        ],
        out_specs=pl.BlockSpec((block_n, o), lambda i: (i, 0)),
        out_shape=jax.ShapeDtypeStruct((n, o), jnp.float32),
    )(x, w_t, b.reshape(1, o))


def _in_proj(x, w_t, b):
    n = x.shape[0]

    def body(xr, wr, br, qr, kr, v0r, v1r):
        y = (jnp.dot(xr[...], wr[...], preferred_element_type=jnp.float32)
             + br[...])
        bn = y.shape[0]
        ones = jnp.ones((bn, 1), jnp.float32)
        zer = jnp.zeros((bn, DV - 129), jnp.float32)
        qr[...] = y[:, :D]
        kr[...] = y[:, D:2 * D]
        v0r[...] = jnp.concatenate([y[:, 2 * D:2 * D + 128], ones, zer],
                                   axis=1)
        v1r[...] = jnp.concatenate([y[:, 2 * D + 128:], ones, zer], axis=1)

    bn = 1000
    return pl.pallas_call(
        body,
        grid=(n // bn,),
        in_specs=[
            pl.BlockSpec((bn, D), lambda i: (i, 0)),
            pl.BlockSpec((D, 3 * D), lambda i: (0, 0)),
            pl.BlockSpec((1, 3 * D), lambda i: (0, 0)),
        ],
        out_specs=[
            pl.BlockSpec((bn, D), lambda i: (i, 0)),
            pl.BlockSpec((bn, D), lambda i: (i, 0)),
            pl.BlockSpec((bn, DV), lambda i: (i, 0)),
            pl.BlockSpec((bn, DV), lambda i: (i, 0)),
        ],
        out_shape=[
            jax.ShapeDtypeStruct((n, D), jnp.float32),
            jax.ShapeDtypeStruct((n, D), jnp.float32),
            jax.ShapeDtypeStruct((n, DV), jnp.float32),
            jax.ShapeDtypeStruct((n, DV), jnp.float32),
        ],
    )(x, w_t, b.reshape(1, 3 * D))


def _out_proj(a0, a1, w_t, b):
    n = a0.shape[0]
    w0t = w_t[:128]
    w1t = w_t[128:]

    def body(a0r, a1r, w0r, w1r, br, yr):
        x0 = a0r[:, :128]
        x1 = a1r[:, :128]
        den = a0r[:, 128:129] + 1e-12
        y = (jnp.dot(x0, w0r[...], preferred_element_type=jnp.float32)
             + jnp.dot(x1, w1r[...], preferred_element_type=jnp.float32))
        yr[...] = y / den + br[...]

    bn = 1000
    return pl.pallas_call(
        body,
        grid=(n // bn,),
        in_specs=[
            pl.BlockSpec((bn, DV), lambda i: (i, 0)),
            pl.BlockSpec((bn, DV), lambda i: (i, 0)),
            pl.BlockSpec((128, D), lambda i: (0, 0)),
            pl.BlockSpec((128, D), lambda i: (0, 0)),
            pl.BlockSpec((1, D), lambda i: (0, 0)),
        ],
        out_specs=pl.BlockSpec((bn, D), lambda i: (i, 0)),
        out_shape=jax.ShapeDtypeStruct((n, D), jnp.float32),
    )(a0, a1, w0t, w1t, b.reshape(1, D))


@functools.partial(
    pl.kernel,
    out_type=jax.ShapeDtypeStruct((NC, N, DV), jnp.float32),
    mesh=_mesh,
    scratch_types=[
        pltpu.VMEM((CHUNK,), jnp.int32),
        pltpu.VMEM((CHUNK,), jnp.int32),
        pltpu.VMEM((CHUNK,), jnp.float32),
        pltpu.VMEM((CHUNK, DV), jnp.float32),
        pltpu.VMEM((125, DV), jnp.float32),
        pltpu.VMEM_SHARED((N, DV), jnp.float32),
        pltpu.SemaphoreType.DMA,
    ],
    compiler_params=_sc_params,
)
def _edge_scatter(v0_hbm, v1_hbm, src_hbm, dst_hbm, ex_hbm, att_hbm,
                  src_v, dst_v, exb, vbuf, zbuf, acc, sem):
    c = lax.axis_index("c")
    s = lax.axis_index("s")

    def zrow(r, carry):
        for j in range(DV // L):
            zbuf[r, pl.ds(j * L, L)] = jnp.zeros((L,), jnp.float32)
        return carry

    lax.fori_loop(0, 125, zrow, 0)
    for t in range(5):
        pltpu.sync_copy(zbuf, acc.at[pl.ds(s * 625 + t * 125, 125)])
    plsc.subcore_barrier()

    def chunk_body(j, carry):
        chunk = s + j * NS

        @pl.when(chunk < NCHUNK)
        def _():
            base = chunk * CHUNK
            pltpu.sync_copy(src_hbm.at[pl.ds(base, CHUNK)], src_v)
            pltpu.sync_copy(dst_hbm.at[pl.ds(base, CHUNK)], dst_v)
            pltpu.sync_copy(ex_hbm.at[pl.ds(base, CHUNK)], exb)

            @pl.when(c == 0)
            def _():
                pltpu.async_copy(v0_hbm.at[src_v], vbuf, sem).wait()

            @pl.when(c == 1)
            def _():
                pltpu.async_copy(v1_hbm.at[src_v], vbuf, sem).wait()

            def edge_group(g, carry2):
                exv = exb[pl.ds(g * L, L)]

                def edge_body(i, carry3):
                    row = g * L + i
                    w = exv.at[jnp.full((L,), i, jnp.int32)].get(
                        mode="promise_in_bounds")
                    for jc in range(DV // L):
                        vbuf[row, pl.ds(jc * L, L)] = (
                            vbuf[row, pl.ds(jc * L, L)] * w)
                    return carry3

                lax.fori_loop(0, L, edge_body, 0)
                return carry2

            lax.fori_loop(0, CHUNK // L, edge_group, 0)
            pltpu.sync_copy(vbuf, acc.at[dst_v], add=True)

        return carry

    lax.fori_loop(0, K3_ITERS, chunk_body, 0)
    plsc.subcore_barrier()
    for t in range(5):
        r0 = s * 625 + t * 125
        pltpu.sync_copy(acc.at[pl.ds(r0, 125)],
                        att_hbm.at[c, pl.ds(r0, 125)])


@functools.partial(
    pl.kernel,
    out_type=jax.ShapeDtypeStruct((E,), jnp.float32),
    mesh=_mesh,
    scratch_types=[
        pltpu.VMEM((CHUNK,), jnp.int32),
        pltpu.VMEM((CHUNK,), jnp.int32),
        pltpu.VMEM((CHUNK, D), jnp.float32),
        pltpu.VMEM((CHUNK, D), jnp.float32),
        pltpu.VMEM((CHUNK,), jnp.float32),
        pltpu.SemaphoreType.DMA,
        pltpu.SemaphoreType.DMA,
    ],
    compiler_params=_sc_params,
)
def _edge_exp(q_hbm, k_hbm, src_hbm, dst_hbm, ex_hbm,
              src_v, dst_v, qbuf, kbuf, exbuf, sem1, sem2):
    wid = lax.axis_index("s") * NC + lax.axis_index("c")
    iota = lax.iota(jnp.int32, L)
    perms = [iota ^ sh for sh in (8, 4, 2, 1)]

    def _lane_sum(v):
        for p in perms:
            v = v + v.at[p].get(mode="promise_in_bounds")
        return v

    def chunk_body(j, carry):
        chunk = wid + j * NW

        @pl.when(chunk < NCHUNK)
        def _():
            base = chunk * CHUNK
            pltpu.sync_copy(src_hbm.at[pl.ds(base, CHUNK)], src_v)
            pltpu.sync_copy(dst_hbm.at[pl.ds(base, CHUNK)], dst_v)
            cp1 = pltpu.async_copy(q_hbm.at[src_v], qbuf, sem1)
            cp2 = pltpu.async_copy(k_hbm.at[dst_v], kbuf, sem2)
            cp1.wait()
            cp2.wait()

            def edge_group(g, carry2):
                def edge_body(i, dots):
                    row = g * L + i
                    acc = jnp.zeros((L,), jnp.float32)
                    for jj in range(D // L):
                        acc = acc + (qbuf[row, pl.ds(jj * L, L)]
                                     * kbuf[row, pl.ds(jj * L, L)])
                    tot = _lane_sum(acc)
                    return jnp.where(iota == i, tot, dots)

                dots = lax.fori_loop(0, L, edge_body,
                                     jnp.zeros((L,), jnp.float32))
                exbuf[pl.ds(g * L, L)] = jnp.exp(dots * (1.0 / 16.0))
                return carry2

            lax.fori_loop(0, CHUNK // L, edge_group, 0)
            pltpu.sync_copy(exbuf, ex_hbm.at[pl.ds(base, CHUNK)])

        return carry

    lax.fori_loop(0, K2_ITERS, chunk_body, 0)


MP = 1024
RCHUNK = 80
NRCHUNK = N // RCHUNK
K5_ITERS = (NRCHUNK + NS - 1) // NS
FPP = 16
NPASS = 128 // FPP


@functools.partial(
    pl.kernel,
    out_type=jax.ShapeDtypeStruct((NC, NPASS, MP, FPP), jnp.float32),
    mesh=_mesh,
    scratch_types=[
        pltpu.VMEM((RCHUNK,), jnp.int32),
        pltpu.VMEM((RCHUNK, FPP), jnp.float32),
        pltpu.VMEM((MP, FPP), jnp.float32),
        pltpu.VMEM((MP, FPP), jnp.float32),
        pltpu.VMEM((MP, L), jnp.float32),
        pltpu.VMEM((64, FPP), jnp.float32),
        pltpu.VMEM((64, L), jnp.float32),
        pltpu.VMEM((64, FPP), jnp.float32),
        pltpu.VMEM((64, FPP), jnp.float32),
        pltpu.VMEM((64, L), jnp.float32),
        pltpu.VMEM((64, FPP), jnp.float32),
        pltpu.VMEM_SHARED((NS, MP, FPP), jnp.float32),
        pltpu.VMEM_SHARED((NS, MP, FPP), jnp.float32),
        pltpu.VMEM_SHARED((NS, MP, L), jnp.float32),
    ],
    compiler_params=_sc_params,
)
def _pool(f_hbm, inv_hbm, pool_hbm,
          invb, rbuf, lsum, lmax, lcnt, tbuf, tcnt, sacc, macc, cacc, obuf,
          psum, pmax, pcnt):
    c = lax.axis_index("c")
    s = lax.axis_index("s")
    iota = lax.iota(jnp.int32, L)
    neginf = jnp.full((L,), -jnp.inf, jnp.float32)
    zeros = jnp.zeros((L,), jnp.float32)
    ones = jnp.ones((L,), jnp.float32)

    for p in range(NPASS):
        def initf(i, carry):
            for fg in range(FPP // L):
                lsum[i, pl.ds(fg * L, L)] = zeros
                lmax[i, pl.ds(fg * L, L)] = neginf
            lcnt[i, pl.ds(0, L)] = zeros
            return carry

        lax.fori_loop(0, MP, initf, 0)

        def rchunk_body(j, carry):
            chunk = s + j * NS

            @pl.when(chunk < NRCHUNK)
            def _():
                r0 = chunk * RCHUNK
                f0 = c * 128 + p * FPP
                pltpu.sync_copy(inv_hbm.at[pl.ds(r0, RCHUNK)], invb)
                pltpu.sync_copy(
                    f_hbm.at[pl.ds(r0, RCHUNK), pl.ds(f0, FPP)], rbuf)

                def row_group(g, carry2):
                    cvec = invb[pl.ds(g * L, L)]
                    for i in range(L):
                        col = cvec[i]
                        r = g * L + i
                        lcnt[col, pl.ds(0, L)] = (lcnt[col, pl.ds(0, L)]
                                                  + ones)
                        for fg in range(FPP // L):
                            vals = rbuf[r, pl.ds(fg * L, L)]
                            lsum[col, pl.ds(fg * L, L)] = (
                                lsum[col, pl.ds(fg * L, L)] + vals)
                            lmax[col, pl.ds(fg * L, L)] = jnp.maximum(
                                lmax[col, pl.ds(fg * L, L)], vals)
                    return carry2

                lax.fori_loop(0, RCHUNK // L, row_group, 0)

            return carry

        lax.fori_loop(0, K5_ITERS, rchunk_body, 0)

        pltpu.sync_copy(lsum, psum.at[s])
        pltpu.sync_copy(lmax, pmax.at[s])
        pltpu.sync_copy(lcnt, pcnt.at[s])
        plsc.subcore_barrier()

        m0 = s * 64

        def initcb(i, carry):
            for fg in range(FPP // L):
                sacc[i, pl.ds(fg * L, L)] = zeros
                macc[i, pl.ds(fg * L, L)] = neginf
            cacc[i, pl.ds(0, L)] = zeros
            return carry

        lax.fori_loop(0, 64, initcb, 0)

        def comb_t(t, carry):
            pltpu.sync_copy(psum.at[t, pl.ds(m0, 64)], tbuf)

            def accs(i, carry2):
                for fg in range(FPP // L):
                    sacc[i, pl.ds(fg * L, L)] = (
                        sacc[i, pl.ds(fg * L, L)]
                        + tbuf[i, pl.ds(fg * L, L)])
                return carry2

            lax.fori_loop(0, 64, accs, 0)
            pltpu.sync_copy(pmax.at[t, pl.ds(m0, 64)], tbuf)

            def accm(i, carry2):
                for fg in range(FPP // L):
                    macc[i, pl.ds(fg * L, L)] = jnp.maximum(
                        macc[i, pl.ds(fg * L, L)],
                        tbuf[i, pl.ds(fg * L, L)])
                return carry2

            lax.fori_loop(0, 64, accm, 0)
            pltpu.sync_copy(pcnt.at[t, pl.ds(m0, 64)], tcnt)

            def accc(i, carry2):
                cacc[i, pl.ds(0, L)] = (cacc[i, pl.ds(0, L)]
                                        + tcnt[i, pl.ds(0, L)])
                return carry2

            lax.fori_loop(0, 64, accc, 0)
            return carry

        lax.fori_loop(0, NS, comb_t, 0)

        def fin_col(i, carry):
            cv = cacc[i, pl.ds(0, L)]
            has = cv > 0.0
            inv_cnt = 1.0 / jnp.maximum(cv, 1.0)
            for fg in range(FPP // L):
                maxv = macc[i, pl.ds(fg * L, L)]
                sumv = sacc[i, pl.ds(fg * L, L)]
                obuf[i, pl.ds(fg * L, L)] = (
                    jnp.where(has, maxv, 0.0) + sumv * inv_cnt)
            return carry

        lax.fori_loop(0, 64, fin_col, 0)

        pltpu.sync_copy(obuf, pool_hbm.at[c, p, pl.ds(m0, 64)])
        plsc.subcore_barrier()


def kernel(x_feat, kernel_map, inverse_map, coor, in_proj_w, in_proj_b,
           out_proj_w, out_proj_b):
    src = kernel_map[0]
    dst = kernel_map[1]
    q, k, v0, v1 = _in_proj(x_feat, in_proj_w.T, in_proj_b)
    ex = _edge_exp(q, k, src, dst)
    att = _edge_scatter(v0, v1, src, dst, ex)
    attended_feat = _out_proj(att[0], att[1], out_proj_w.T, out_proj_b)
    pool = _pool(attended_feat, inverse_map)
    out = pool.transpose(2, 0, 1, 3).reshape(MP, D)[:M]
    return (coor, out)


# speedup vs baseline: 1.2722x; 1.2722x over previous
import functools

import jax
import jax.numpy as jnp
import numpy as np
from jax import lax
from jax.experimental import pallas as pl
from jax.experimental.pallas import tpu as pltpu
from jax.experimental.pallas import tpu_sc as plsc

N = 10000
E = 160000
M = 1000
D = 256
EMBED_DIM = 256
NC, NS, L = 2, 16, 16
NW = NC * NS

DV = 144

CHUNK = 128
NCHUNK = E // CHUNK
K2_ITERS = (NCHUNK + NW - 1) // NW
K3_ITERS = (NCHUNK + NS - 1) // NS

_mesh = plsc.VectorSubcoreMesh(core_axis_name="c", subcore_axis_name="s")
_sc_params = pltpu.CompilerParams(use_tc_tiling_on_sc=False)


def _matmul_bias(x, w_t, b, block_n):
    n, k = x.shape
    o = w_t.shape[1]

    def body(xr, wr, br, yr):
        yr[...] = (
            jnp.dot(xr[...], wr[...], preferred_element_type=jnp.float32)
            + br[...]
        )

    return pl.pallas_call(
        body,
        grid=(n // block_n,),
        in_specs=[
            pl.BlockSpec((block_n, k), lambda i: (i, 0)),
            pl.BlockSpec((k, o), lambda i: (0, 0)),
            pl.BlockSpec((1, o), lambda i: (0, 0)),
        ],
        out_specs=pl.BlockSpec((block_n, o), lambda i: (i, 0)),
        out_shape=jax.ShapeDtypeStruct((n, o), jnp.float32),
    )(x, w_t, b.reshape(1, o))


def _in_proj(x, w_t, b):
    n = x.shape[0]

    def body(xr, wr, br, qr, kr, v0r, v1r):
        y = (jnp.dot(xr[...], wr[...], preferred_element_type=jnp.float32)
             + br[...])
        bn = y.shape[0]
        ones = jnp.ones((bn, 1), jnp.float32)
        zer = jnp.zeros((bn, DV - 129), jnp.float32)
        qr[...] = y[:, :D]
        kr[...] = y[:, D:2 * D]
        v0r[...] = jnp.concatenate([y[:, 2 * D:2 * D + 128], ones, zer],
                                   axis=1)
        v1r[...] = jnp.concatenate([y[:, 2 * D + 128:], ones, zer], axis=1)

    bn = 1000
    return pl.pallas_call(
        body,
        grid=(n // bn,),
        in_specs=[
            pl.BlockSpec((bn, D), lambda i: (i, 0)),
            pl.BlockSpec((D, 3 * D), lambda i: (0, 0)),
            pl.BlockSpec((1, 3 * D), lambda i: (0, 0)),
        ],
        out_specs=[
            pl.BlockSpec((bn, D), lambda i: (i, 0)),
            pl.BlockSpec((bn, D), lambda i: (i, 0)),
            pl.BlockSpec((bn, DV), lambda i: (i, 0)),
            pl.BlockSpec((bn, DV), lambda i: (i, 0)),
        ],
        out_shape=[
            jax.ShapeDtypeStruct((n, D), jnp.float32),
            jax.ShapeDtypeStruct((n, D), jnp.float32),
            jax.ShapeDtypeStruct((n, DV), jnp.float32),
            jax.ShapeDtypeStruct((n, DV), jnp.float32),
        ],
    )(x, w_t, b.reshape(1, 3 * D))


def _out_proj(a0, a1, w_t, b):
    n = a0.shape[0]
    w0t = w_t[:128]
    w1t = w_t[128:]

    def body(a0r, a1r, w0r, w1r, br, yr):
        x0 = a0r[:, :128]
        x1 = a1r[:, :128]
        den = a0r[:, 128:129] + 1e-12
        y = (jnp.dot(x0, w0r[...], preferred_element_type=jnp.float32)
             + jnp.dot(x1, w1r[...], preferred_element_type=jnp.float32))
        yr[...] = y / den + br[...]

    bn = 1000
    return pl.pallas_call(
        body,
        grid=(n // bn,),
        in_specs=[
            pl.BlockSpec((bn, DV), lambda i: (i, 0)),
            pl.BlockSpec((bn, DV), lambda i: (i, 0)),
            pl.BlockSpec((128, D), lambda i: (0, 0)),
            pl.BlockSpec((128, D), lambda i: (0, 0)),
            pl.BlockSpec((1, D), lambda i: (0, 0)),
        ],
        out_specs=pl.BlockSpec((bn, D), lambda i: (i, 0)),
        out_shape=jax.ShapeDtypeStruct((n, D), jnp.float32),
    )(a0, a1, w0t, w1t, b.reshape(1, D))


@functools.partial(
    pl.kernel,
    out_type=jax.ShapeDtypeStruct((NC, N, DV), jnp.float32),
    mesh=_mesh,
    scratch_types=[
        pltpu.VMEM((CHUNK,), jnp.int32),
        pltpu.VMEM((CHUNK,), jnp.int32),
        pltpu.VMEM((CHUNK,), jnp.float32),
        pltpu.VMEM((CHUNK, DV), jnp.float32),
        pltpu.VMEM((125, DV), jnp.float32),
        pltpu.VMEM_SHARED((N, DV), jnp.float32),
        pltpu.SemaphoreType.DMA,
    ],
    compiler_params=_sc_params,
)
def _edge_scatter(v0_hbm, v1_hbm, src_hbm, dst_hbm, ex_hbm, att_hbm,
                  src_v, dst_v, exb, vbuf, zbuf, acc, sem):
    c = lax.axis_index("c")
    s = lax.axis_index("s")

    def zrow(r, carry):
        for j in range(DV // L):
            zbuf[r, pl.ds(j * L, L)] = jnp.zeros((L,), jnp.float32)
        return carry

    lax.fori_loop(0, 125, zrow, 0)
    for t in range(5):
        pltpu.sync_copy(zbuf, acc.at[pl.ds(s * 625 + t * 125, 125)])
    plsc.subcore_barrier()

    def chunk_body(j, carry):
        chunk = s + j * NS

        @pl.when(chunk < NCHUNK)
        def _():
            base = chunk * CHUNK
            pltpu.sync_copy(src_hbm.at[pl.ds(base, CHUNK)], src_v)
            pltpu.sync_copy(dst_hbm.at[pl.ds(base, CHUNK)], dst_v)
            pltpu.sync_copy(ex_hbm.at[pl.ds(base, CHUNK)], exb)

            @pl.when(c == 0)
            def _():
                pltpu.async_copy(v0_hbm.at[src_v], vbuf, sem).wait()

            @pl.when(c == 1)
            def _():
                pltpu.async_copy(v1_hbm.at[src_v], vbuf, sem).wait()

            def edge_group(g, carry2):
                exv = exb[pl.ds(g * L, L)]

                def edge_body(i, carry3):
                    row = g * L + i
                    w = exv.at[jnp.full((L,), i, jnp.int32)].get(
                        mode="promise_in_bounds")
                    for jc in range(DV // L):
                        vbuf[row, pl.ds(jc * L, L)] = (
                            vbuf[row, pl.ds(jc * L, L)] * w)
                    return carry3

                lax.fori_loop(0, L, edge_body, 0)
                return carry2

            lax.fori_loop(0, CHUNK // L, edge_group, 0)
            pltpu.sync_copy(vbuf, acc.at[dst_v], add=True)

        return carry

    lax.fori_loop(0, K3_ITERS, chunk_body, 0)
    plsc.subcore_barrier()
    for t in range(5):
        r0 = s * 625 + t * 125
        pltpu.sync_copy(acc.at[pl.ds(r0, 125)],
                        att_hbm.at[c, pl.ds(r0, 125)])


@functools.partial(
    pl.kernel,
    out_type=jax.ShapeDtypeStruct((E,), jnp.float32),
    mesh=_mesh,
    scratch_types=[
        pltpu.VMEM((CHUNK,), jnp.int32),
        pltpu.VMEM((CHUNK,), jnp.int32),
        pltpu.VMEM((CHUNK, D), jnp.float32),
        pltpu.VMEM((CHUNK, D), jnp.float32),
        pltpu.VMEM((CHUNK,), jnp.float32),
        pltpu.SemaphoreType.DMA,
        pltpu.SemaphoreType.DMA,
    ],
    compiler_params=_sc_params,
)
def _edge_exp(q_hbm, k_hbm, src_hbm, dst_hbm, ex_hbm,
              src_v, dst_v, qbuf, kbuf, exbuf, sem1, sem2):
    wid = lax.axis_index("s") * NC + lax.axis_index("c")
    iota = lax.iota(jnp.int32, L)
    perms = [iota ^ sh for sh in (8, 4, 2, 1)]

    def _lane_sum(v):
        for p in perms:
            v = v + v.at[p].get(mode="promise_in_bounds")
        return v

    def chunk_body(j, carry):
        chunk = wid + j * NW

        @pl.when(chunk < NCHUNK)
        def _():
            base = chunk * CHUNK
            pltpu.sync_copy(src_hbm.at[pl.ds(base, CHUNK)], src_v)
            pltpu.sync_copy(dst_hbm.at[pl.ds(base, CHUNK)], dst_v)
            cp1 = pltpu.async_copy(q_hbm.at[src_v], qbuf, sem1)
            cp2 = pltpu.async_copy(k_hbm.at[dst_v], kbuf, sem2)
            cp1.wait()
            cp2.wait()

            def edge_group(g, carry2):
                def edge_body(i, dots):
                    row = g * L + i
                    acc = jnp.zeros((L,), jnp.float32)
                    for jj in range(D // L):
                        acc = acc + (qbuf[row, pl.ds(jj * L, L)]
                                     * kbuf[row, pl.ds(jj * L, L)])
                    tot = _lane_sum(acc)
                    return jnp.where(iota == i, tot, dots)

                dots = lax.fori_loop(0, L, edge_body,
                                     jnp.zeros((L,), jnp.float32))
                exbuf[pl.ds(g * L, L)] = jnp.exp(dots * (1.0 / 16.0))
                return carry2

            lax.fori_loop(0, CHUNK // L, edge_group, 0)
            pltpu.sync_copy(exbuf, ex_hbm.at[pl.ds(base, CHUNK)])

        return carry

    lax.fori_loop(0, K2_ITERS, chunk_body, 0)


MP = 1024
RCHUNK = 80
NRCHUNK = N // RCHUNK
K5_ITERS = (NRCHUNK + NS - 1) // NS
FPP = 32
NPASS = 128 // FPP


@functools.partial(
    pl.kernel,
    out_type=[
        jax.ShapeDtypeStruct((NC, NPASS, MP, FPP), jnp.float32),
        jax.ShapeDtypeStruct((NC, NS, MP, FPP), jnp.float32),
    ],
    mesh=_mesh,
    scratch_types=[
        pltpu.VMEM((RCHUNK,), jnp.int32),
        pltpu.VMEM((RCHUNK, DV), jnp.float32),
        pltpu.VMEM((RCHUNK, FPP), jnp.float32),
        pltpu.VMEM((MP, FPP), jnp.float32),
        pltpu.VMEM((NS, 64, FPP), jnp.float32),
        pltpu.VMEM((64, DV), jnp.float32),
        pltpu.VMEM((64, FPP), jnp.float32),
        pltpu.VMEM_SHARED((MP, DV), jnp.float32),
    ],
    compiler_params=_sc_params,
)
def _pool(f_hbm, inv_hbm, pool_hbm, pmax,
          invb, rbufa, rbufp, lmax, tmax, pbuf, obuf, accp):
    c = lax.axis_index("c")
    s = lax.axis_index("s")
    iota = lax.iota(jnp.int32, L)
    neginf = jnp.full((L,), -jnp.inf, jnp.float32)
    zeros = jnp.zeros((L,), jnp.float32)
    e0 = jnp.where(iota == 0, 1.0, 0.0).astype(jnp.float32)
    m0 = s * 64

    def initr(r, carry):
        rbufa[r, pl.ds(128, L)] = e0
        return carry

    lax.fori_loop(0, RCHUNK, initr, 0)

    def zacc(i, carry):
        for jg in range(DV // L):
            pbuf[i, pl.ds(jg * L, L)] = zeros
        return carry

    lax.fori_loop(0, 64, zacc, 0)
    pltpu.sync_copy(pbuf, accp.at[pl.ds(m0, 64)])
    plsc.subcore_barrier()

    def sum_chunk(j, carry):
        chunk = s + j * NS

        @pl.when(chunk < NRCHUNK)
        def _():
            r0 = chunk * RCHUNK
            pltpu.sync_copy(inv_hbm.at[pl.ds(r0, RCHUNK)], invb)
            pltpu.sync_copy(
                f_hbm.at[pl.ds(r0, RCHUNK), pl.ds(c * 128, 128)],
                rbufa.at[:, pl.ds(0, 128)])
            pltpu.sync_copy(rbufa, accp.at[invb], add=True)

        return carry

    lax.fori_loop(0, K5_ITERS, sum_chunk, 0)

    for p in range(NPASS):
        def initf(i, carry):
            for fg in range(FPP // L):
                lmax[i, pl.ds(fg * L, L)] = neginf
            return carry

        lax.fori_loop(0, MP, initf, 0)

        def rchunk_body(j, carry):
            chunk = s + j * NS

            @pl.when(chunk < NRCHUNK)
            def _():
                r0 = chunk * RCHUNK
                f0 = c * 128 + p * FPP
                pltpu.sync_copy(inv_hbm.at[pl.ds(r0, RCHUNK)], invb)
                pltpu.sync_copy(
                    f_hbm.at[pl.ds(r0, RCHUNK), pl.ds(f0, FPP)], rbufp)

                def row_group(g, carry2):
                    cvec = invb[pl.ds(g * L, L)]
                    for i in range(L):
                        col = cvec[i]
                        r = g * L + i
                        for fg in range(FPP // L):
                            vals = rbufp[r, pl.ds(fg * L, L)]
                            lmax[col, pl.ds(fg * L, L)] = jnp.maximum(
                                lmax[col, pl.ds(fg * L, L)], vals)
                    return carry2

                lax.fori_loop(0, RCHUNK // L, row_group, 0)

            return carry

        lax.fori_loop(0, K5_ITERS, rchunk_body, 0)

        pltpu.sync_copy(lmax, pmax.at[c, s])
        plsc.subcore_barrier()

        pltpu.sync_copy(pmax.at[c, :, pl.ds(m0, 64)], tmax)
        if p == 0:
            pltpu.sync_copy(accp.at[pl.ds(m0, 64)], pbuf)

        def fin_col(i, carry):
            crow = pbuf[i, pl.ds(128, L)]
            cnt = crow[0]
            cdiv = jnp.maximum(jnp.full((L,), cnt, jnp.float32), 1.0)
            for fg in range(FPP // L):
                mv = tmax[0, i, pl.ds(fg * L, L)]
                for t in range(1, NS):
                    mv = jnp.maximum(mv, tmax[t, i, pl.ds(fg * L, L)])
                sumv = pbuf[i, pl.ds(p * FPP + fg * L, L)]
                obuf[i, pl.ds(fg * L, L)] = mv + sumv / cdiv

            @pl.when(cnt <= 0.0)
            def _():
                for fg in range(FPP // L):
                    obuf[i, pl.ds(fg * L, L)] = zeros

            return carry

        lax.fori_loop(0, 64, fin_col, 0)

        pltpu.sync_copy(obuf, pool_hbm.at[c, p, pl.ds(m0, 64)])
        plsc.subcore_barrier()


def kernel(x_feat, kernel_map, inverse_map, coor, in_proj_w, in_proj_b,
           out_proj_w, out_proj_b):
    src = kernel_map[0]
    dst = kernel_map[1]
    q, k, v0, v1 = _in_proj(x_feat, in_proj_w.T, in_proj_b)
    ex = _edge_exp(q, k, src, dst)
    att = _edge_scatter(v0, v1, src, dst, ex)
    attended_feat = _out_proj(att[0], att[1], out_proj_w.T, out_proj_b)
    pool, _ = _pool(attended_feat, inverse_map)
    out = pool.transpose(2, 0, 1, 3).reshape(MP, D)[:M]
    return (coor, out)
